# baseline (speedup 1.0000x reference)
import functools

import jax
import jax.numpy as jnp
import numpy as np
from jax import lax
from jax.experimental import pallas as pl
from jax.experimental.pallas import tpu as pltpu
from jax.experimental.pallas import tpu_sc as plsc

_F = 26
_D = 16
_B = 4096
_NC, _NS = 2, 16
_NW = _NC * _NS
_RPW = _B // _NW
_CH = 16
_NCH = _RPW // _CH
_RPC = _CH * _F
_G = 4
_RPG = _RPC // _G

_OFFSETS = (np.arange(_F, dtype=np.int32) * 100000)[None, :]

_mesh = plsc.VectorSubcoreMesh(core_axis_name="c", subcore_axis_name="s")

_GATHER_DNUMS = lax.GatherDimensionNumbers(
    offset_dims=(), collapsed_slice_dims=(0,), start_index_map=(0,)
)


def _lane_gather(v, idx2d):
    return lax.gather(
        v, idx2d, _GATHER_DNUMS, (1,),
        mode=lax.GatherScatterMode.PROMISE_IN_BOUNDS,
    )


def _body(emb_hbm, fcw_hbm, bias_hbm, idxe_hbm, idxf_hbm, out_hbm,
          idxe_v, idxf_v, fc_v, emb_v, bias_v, y_v, sem_f, sem0, sem1):
    wid = lax.axis_index("s") * _NC + lax.axis_index("c")

    pltpu.sync_copy(idxe_hbm.at[wid], idxe_v)
    pltpu.sync_copy(idxf_hbm.at[wid], idxf_v)
    pltpu.sync_copy(bias_hbm, bias_v)

    for f in range(_F):
        pltpu.make_async_copy(
            fcw_hbm.at[idxf_v.at[f]], fc_v.at[pl.ds(f * _RPW, _RPW)], sem_f
        ).start()

    def issue_emb(c, slot, sem):
        for g in range(_G):
            pltpu.make_async_copy(
                emb_hbm.at[idxe_v.at[c, g]],
                emb_v.at[slot, pl.ds(g * _RPG, _RPG)],
                sem,
            ).start()

    def wait_emb(sem):
        for g in range(_G):
            pltpu.make_async_copy(
                emb_hbm.at[pl.ds(0, _RPG), :],
                emb_v.at[0, pl.ds(g * _RPG, _RPG)],
                sem,
            ).wait()

    issue_emb(0, 0, sem0)
    for f in range(_F):
        pltpu.make_async_copy(
            fcw_hbm.at[pl.ds(0, _RPW)], fc_v.at[pl.ds(f * _RPW, _RPW)], sem_f
        ).wait()

    lanes = lax.iota(jnp.int32, _D)
    perms = [(lanes ^ sh)[:, None] for sh in (8, 4, 2, 1)]

    def compute_chunk(c, slot):
        def row_body(r, inter):
            p = r * _F
            z = jnp.zeros((_D,), jnp.float32)
            s1l = z
            s2l = z
            s1h = z
            s2h = z
            s3h = z
            for f in range(_F):
                vlo = emb_v[slot, p + f, pl.ds(0, _D)]
                vhi = emb_v[slot, p + f, pl.ds(_D, _D)]
                s1l = s1l + vlo
                s2l = s2l + vlo * vlo
                sq = vhi * vhi
                s1h = s1h + vhi
                s2h = s2h + sq
                s3h = s3h + sq * vhi
            w = 0.5 * (s1l * s1l - s2l) + (1.0 / 6.0) * (
                s1h * (s1h * s1h - 3.0 * s2h) + 2.0 * s3h
            )
            for pm in perms:
                w = w + _lane_gather(w, pm)
            return jnp.where(lanes == r, w, inter)

        inter = lax.fori_loop(0, _CH, row_body, jnp.zeros((_D,), jnp.float32))
        lin = bias_v[...]
        for f in range(_F):
            lin = lin + fc_v[pl.ds(f * _RPW + c * _CH, _CH)]
        y = lin + inter
        y_v[pl.ds(c * _CH, _CH)] = 1.0 / (1.0 + jnp.exp(-y))

    def pair_body(i, carry):
        c0 = 2 * i
        issue_emb(c0 + 1, 1, sem1)
        wait_emb(sem0)
        compute_chunk(c0, 0)

        @pl.when(c0 + 2 < _NCH)
        def _():
            issue_emb(c0 + 2, 0, sem0)

        wait_emb(sem1)
        compute_chunk(c0 + 1, 1)
        return carry

    lax.fori_loop(0, _NCH // 2, pair_body, 0)

    pltpu.sync_copy(y_v, out_hbm.at[pl.ds(wid * _RPW, _RPW)])


_fm_kernel = functools.partial(
    pl.kernel,
    out_type=jax.ShapeDtypeStruct((_B,), jnp.float32),
    mesh=_mesh,
    scratch_types=[
        pltpu.VMEM((_NCH, _G, _RPG), jnp.int32),
        pltpu.VMEM((_F, _RPW), jnp.int32),
        pltpu.VMEM((_F * _RPW,), jnp.float32),
        pltpu.VMEM((2, _RPC, 2 * _D), jnp.float32),
        pltpu.VMEM((_D,), jnp.float32),
        pltpu.VMEM((_RPW,), jnp.float32),
        pltpu.SemaphoreType.DMA,
        pltpu.SemaphoreType.DMA,
        pltpu.SemaphoreType.DMA,
    ],
    compiler_params=pltpu.CompilerParams(use_tc_tiling_on_sc=False),
)(_body)


@jax.jit
def kernel(x, fc_weight, fc_bias, emb_weight):
    xo = x.astype(jnp.int32) + jnp.asarray(_OFFSETS)
    idxe = xo.reshape(_NW, _NCH, _G, _RPG)
    idxf = jnp.transpose(xo.reshape(_NW, _RPW, _F), (0, 2, 1))
    fcw = fc_weight.reshape(-1)
    bias16 = jnp.broadcast_to(fc_bias.astype(jnp.float32), (_D,))
    return _fm_kernel(emb_weight, fcw, bias16, idxe, idxf)
